# baseline (speedup 1.0000x reference)
import jax
import jax.numpy as jnp
from jax.experimental import pallas as pl
from jax.experimental.pallas import tpu as pltpu

B, H, N = 4, 12, 577
NT = N - 1
K = int(N * 0.5)
HPB = 1
NHB = H // HPB


def _fused_kernel(v3_ref, x2_ref, x1_ref, x0_ref, mask_ref, vals_ref, t_scr):
    b = pl.program_id(0)
    hb = pl.program_id(1)
    for hh in range(HPB):
        p1 = jnp.dot(x1_ref[0, hh], x0_ref[0, hh],
                     preferred_element_type=jnp.float32)
        p2 = jnp.dot(x2_ref[0, hh], p1,
                     preferred_element_type=jnp.float32)
        v = jnp.dot(v3_ref[b, hb * HPB + hh], p2,
                    preferred_element_type=jnp.float32)
        t_scr[hb * HPB + hh, b] = v[:, 1:]

    @pl.when((b == B - 1) & (hb == NHB - 1))
    def _():
        t3 = t_scr[:, :, 0, :]
        bits = jax.lax.bitcast_convert_type(t3, jnp.int32)

        def bit_body(i, prefix):
            cand = prefix | (jnp.int32(1) << (jnp.int32(30) - i))
            ge = (bits >= cand).astype(jnp.float32)
            cnt = jnp.sum(ge, axis=2, keepdims=True)
            return jnp.where(cnt >= float(K), cand, prefix)

        thr = jax.lax.fori_loop(
            0, 31, bit_body, jnp.zeros((H, B, 1), dtype=jnp.int32))

        gt = (bits > thr).astype(jnp.float32)
        eq = bits == thr
        n_gt = jnp.sum(gt, axis=2, keepdims=True)
        need = float(K) - n_gt

        jj = jax.lax.broadcasted_iota(jnp.int32, (H, B, NT), 2)

        def idx_body(i, bound):
            cand = bound | (jnp.int32(1) << (jnp.int32(9) - i))
            take = jnp.where(eq & (jj < cand), 1.0, 0.0)
            cnt = jnp.sum(take, axis=2, keepdims=True)
            return jnp.where(cnt >= need, bound, cand)

        idx_bound = jax.lax.fori_loop(
            0, 10, idx_body, jnp.zeros((H, B, 1), dtype=jnp.int32))
        sel = jnp.maximum(gt, jnp.where(eq & (jj < idx_bound), 1.0, 0.0))

        mask_ref[...] = jnp.max(sel, axis=0)

        t11 = t3[H - 1]
        a = t11[:, :, None]
        bvals = t11[:, None, :]
        ii = jax.lax.broadcasted_iota(jnp.int32, (B, NT, NT), 1)
        jj2 = jax.lax.broadcasted_iota(jnp.int32, (B, NT, NT), 2)
        beats = (a > bvals) | ((a == bvals) & (ii < jj2))
        rank = jnp.sum(beats.astype(jnp.float32), axis=1)
        rr = jax.lax.broadcasted_iota(
            jnp.int32, (B, NT, K), 2).astype(jnp.float32)
        onehot = jnp.where(rank[:, :, None] == rr, 1.0, 0.0)
        vals_ref[...] = jnp.sum(t11[:, :, None] * onehot, axis=1)


def kernel(x0, x1, x2, x3):
    v3 = x3[:, :, 0:1, :]

    mask_f32, vals = pl.pallas_call(
        _fused_kernel,
        grid=(B, NHB),
        in_specs=[
            pl.BlockSpec((B, H, 1, N), lambda b, h: (0, 0, 0, 0)),
            pl.BlockSpec((1, HPB, N, N), lambda b, h: (b, h, 0, 0)),
            pl.BlockSpec((1, HPB, N, N), lambda b, h: (b, h, 0, 0)),
            pl.BlockSpec((1, HPB, N, N), lambda b, h: (b, h, 0, 0)),
        ],
        out_specs=[
            pl.BlockSpec((B, NT), lambda b, h: (0, 0)),
            pl.BlockSpec((B, K), lambda b, h: (0, 0)),
        ],
        out_shape=[
            jax.ShapeDtypeStruct((B, NT), jnp.float32),
            jax.ShapeDtypeStruct((B, K), jnp.float32),
        ],
        scratch_shapes=[pltpu.VMEM((H, B, 1, NT), jnp.float32)],
    )(v3, x2, x1, x0)

    mask = jnp.concatenate(
        [mask_f32 != 0.0, jnp.zeros((B, 1), dtype=bool)], axis=1)
    return vals, mask


# speedup vs baseline: 1.0502x; 1.0502x over previous
import jax
import jax.numpy as jnp
from jax.experimental import pallas as pl
from jax.experimental.pallas import tpu as pltpu

B, H, N = 4, 12, 577
NT = N - 1
K = int(N * 0.5)
HPB = 2
NHB = H // HPB


def _fused_kernel(v3_ref, x2_ref, x1_ref, x0_ref, mask_ref, vals_ref, t_scr):
    b = pl.program_id(0)
    hb = pl.program_id(1)
    for hh in range(HPB):
        p1 = jnp.dot(x1_ref[0, hh], x0_ref[0, hh],
                     preferred_element_type=jnp.float32)
        p2 = jnp.dot(x2_ref[0, hh], p1,
                     preferred_element_type=jnp.float32)
        v = jnp.dot(v3_ref[b, hb * HPB + hh], p2,
                    preferred_element_type=jnp.float32)
        t_scr[hb * HPB + hh, b] = v[:, 1:]

    @pl.when((b == B - 1) & (hb == NHB - 1))
    def _():
        t3 = t_scr[:, :, 0, :]
        bits = jax.lax.bitcast_convert_type(t3, jnp.int32)

        def bit_body(i, prefix):
            cand = prefix | (jnp.int32(1) << (jnp.int32(30) - i))
            ge = (bits >= cand).astype(jnp.float32)
            cnt = jnp.sum(ge, axis=2, keepdims=True)
            return jnp.where(cnt >= float(K), cand, prefix)

        thr = jax.lax.fori_loop(
            0, 31, bit_body, jnp.zeros((H, B, 1), dtype=jnp.int32))

        gt = (bits > thr).astype(jnp.float32)
        eq = bits == thr
        n_gt = jnp.sum(gt, axis=2, keepdims=True)
        need = float(K) - n_gt

        jj = jax.lax.broadcasted_iota(jnp.int32, (H, B, NT), 2)

        def idx_body(i, bound):
            cand = bound | (jnp.int32(1) << (jnp.int32(9) - i))
            take = jnp.where(eq & (jj < cand), 1.0, 0.0)
            cnt = jnp.sum(take, axis=2, keepdims=True)
            return jnp.where(cnt >= need, bound, cand)

        idx_bound = jax.lax.fori_loop(
            0, 10, idx_body, jnp.zeros((H, B, 1), dtype=jnp.int32))
        sel = jnp.maximum(gt, jnp.where(eq & (jj < idx_bound), 1.0, 0.0))

        mask_ref[...] = jnp.max(sel, axis=0)

        t11 = t3[H - 1]
        a = t11[:, :, None]
        bvals = t11[:, None, :]
        ii = jax.lax.broadcasted_iota(jnp.int32, (B, NT, NT), 1)
        jj2 = jax.lax.broadcasted_iota(jnp.int32, (B, NT, NT), 2)
        beats = (a > bvals) | ((a == bvals) & (ii < jj2))
        rank = jnp.sum(beats.astype(jnp.float32), axis=1)
        rr = jax.lax.broadcasted_iota(
            jnp.int32, (B, NT, K), 2).astype(jnp.float32)
        onehot = jnp.where(rank[:, :, None] == rr, 1.0, 0.0)
        vals_ref[...] = jnp.sum(t11[:, :, None] * onehot, axis=1)


def kernel(x0, x1, x2, x3):
    v3 = x3[:, :, 0:1, :]

    mask_f32, vals = pl.pallas_call(
        _fused_kernel,
        grid=(B, NHB),
        in_specs=[
            pl.BlockSpec((B, H, 1, N), lambda b, h: (0, 0, 0, 0)),
            pl.BlockSpec((1, HPB, N, N), lambda b, h: (b, h, 0, 0)),
            pl.BlockSpec((1, HPB, N, N), lambda b, h: (b, h, 0, 0)),
            pl.BlockSpec((1, HPB, N, N), lambda b, h: (b, h, 0, 0)),
        ],
        out_specs=[
            pl.BlockSpec((B, NT), lambda b, h: (0, 0)),
            pl.BlockSpec((B, K), lambda b, h: (0, 0)),
        ],
        out_shape=[
            jax.ShapeDtypeStruct((B, NT), jnp.float32),
            jax.ShapeDtypeStruct((B, K), jnp.float32),
        ],
        scratch_shapes=[pltpu.VMEM((H, B, 1, NT), jnp.float32)],
    )(v3, x2, x1, x0)

    mask = jnp.concatenate(
        [mask_f32 != 0.0, jnp.zeros((B, 1), dtype=bool)], axis=1)
    return vals, mask


# speedup vs baseline: 1.0539x; 1.0036x over previous
import jax
import jax.numpy as jnp
from jax.experimental import pallas as pl
from jax.experimental.pallas import tpu as pltpu

B, H, N = 4, 12, 577
NT = N - 1
K = int(N * 0.5)
HPB = 3
NHB = H // HPB


def _fused_kernel(v3_ref, x2_ref, x1_ref, x0_ref, mask_ref, vals_ref, t_scr):
    b = pl.program_id(0)
    hb = pl.program_id(1)
    for hh in range(HPB):
        p1 = jnp.dot(x1_ref[0, hh], x0_ref[0, hh],
                     preferred_element_type=jnp.float32)
        p2 = jnp.dot(x2_ref[0, hh], p1,
                     preferred_element_type=jnp.float32)
        v = jnp.dot(v3_ref[b, hb * HPB + hh], p2,
                    preferred_element_type=jnp.float32)
        t_scr[hb * HPB + hh, b] = v[:, 1:]

    @pl.when((b == B - 1) & (hb == NHB - 1))
    def _():
        t3 = t_scr[:, :, 0, :]
        bits = jax.lax.bitcast_convert_type(t3, jnp.int32)

        def bit_body(i, prefix):
            cand = prefix | (jnp.int32(1) << (jnp.int32(30) - i))
            ge = (bits >= cand).astype(jnp.float32)
            cnt = jnp.sum(ge, axis=2, keepdims=True)
            return jnp.where(cnt >= float(K), cand, prefix)

        thr = jax.lax.fori_loop(
            0, 31, bit_body, jnp.zeros((H, B, 1), dtype=jnp.int32))

        gt = (bits > thr).astype(jnp.float32)
        eq = bits == thr
        n_gt = jnp.sum(gt, axis=2, keepdims=True)
        need = float(K) - n_gt

        jj = jax.lax.broadcasted_iota(jnp.int32, (H, B, NT), 2)

        def idx_body(i, bound):
            cand = bound | (jnp.int32(1) << (jnp.int32(9) - i))
            take = jnp.where(eq & (jj < cand), 1.0, 0.0)
            cnt = jnp.sum(take, axis=2, keepdims=True)
            return jnp.where(cnt >= need, bound, cand)

        idx_bound = jax.lax.fori_loop(
            0, 10, idx_body, jnp.zeros((H, B, 1), dtype=jnp.int32))
        sel = jnp.maximum(gt, jnp.where(eq & (jj < idx_bound), 1.0, 0.0))

        mask_ref[...] = jnp.max(sel, axis=0)

        t11 = t3[H - 1]
        a = t11[:, :, None]
        bvals = t11[:, None, :]
        ii = jax.lax.broadcasted_iota(jnp.int32, (B, NT, NT), 1)
        jj2 = jax.lax.broadcasted_iota(jnp.int32, (B, NT, NT), 2)
        beats = (a > bvals) | ((a == bvals) & (ii < jj2))
        rank = jnp.sum(beats.astype(jnp.float32), axis=1)
        rr = jax.lax.broadcasted_iota(
            jnp.int32, (B, NT, K), 2).astype(jnp.float32)
        onehot = jnp.where(rank[:, :, None] == rr, 1.0, 0.0)
        vals_ref[...] = jnp.sum(t11[:, :, None] * onehot, axis=1)


def kernel(x0, x1, x2, x3):
    v3 = x3[:, :, 0:1, :]

    mask_f32, vals = pl.pallas_call(
        _fused_kernel,
        grid=(B, NHB),
        in_specs=[
            pl.BlockSpec((B, H, 1, N), lambda b, h: (0, 0, 0, 0)),
            pl.BlockSpec((1, HPB, N, N), lambda b, h: (b, h, 0, 0)),
            pl.BlockSpec((1, HPB, N, N), lambda b, h: (b, h, 0, 0)),
            pl.BlockSpec((1, HPB, N, N), lambda b, h: (b, h, 0, 0)),
        ],
        out_specs=[
            pl.BlockSpec((B, NT), lambda b, h: (0, 0)),
            pl.BlockSpec((B, K), lambda b, h: (0, 0)),
        ],
        out_shape=[
            jax.ShapeDtypeStruct((B, NT), jnp.float32),
            jax.ShapeDtypeStruct((B, K), jnp.float32),
        ],
        scratch_shapes=[pltpu.VMEM((H, B, 1, NT), jnp.float32)],
    )(v3, x2, x1, x0)

    mask = jnp.concatenate(
        [mask_f32 != 0.0, jnp.zeros((B, 1), dtype=bool)], axis=1)
    return vals, mask
